# baseline (speedup 1.0000x reference)
import functools

import jax
import jax.numpy as jnp
from jax import lax
from jax.experimental import pallas as pl
from jax.experimental.pallas import tpu as pltpu
from jax.experimental.pallas import tpu_sc as plsc

NSEG = 10000
D = 128
N = 320000
NC = 2
NS = 16
NW = NC * NS
ROWS_PER_TILE = N // NW
CHUNK = 100
NCHUNK = ROWS_PER_TILE // CHUNK
NBUF = 3
SEG_PER_TILE = NSEG // NS
CW = 16
EXCHUNKS = [(0, 100), (100, 100), (200, 100), (300, 100),
            (400, 100), (500, 100), (600, 25)]

_mesh = plsc.VectorSubcoreMesh(core_axis_name="c", subcore_axis_name="s")


@functools.partial(
    pl.kernel,
    mesh=_mesh,
    compiler_params=pltpu.CompilerParams(use_tc_tiling_on_sc=False),
    out_type=[
        jax.ShapeDtypeStruct((NC * NSEG, D), jnp.float32),
        jax.ShapeDtypeStruct((NC * NSEG, CW), jnp.float32),
    ],
    scratch_types=[
        [pltpu.VMEM((1, CHUNK), jnp.int32) for _ in range(NBUF)],
        [pltpu.VMEM((CHUNK, D), jnp.float32) for _ in range(NBUF)],
        pltpu.VMEM((CHUNK, CW), jnp.float32),
        pltpu.VMEM_SHARED((NSEG, D), jnp.float32),
        pltpu.VMEM_SHARED((NSEG, CW), jnp.float32),
        [pltpu.SemaphoreType.DMA for _ in range(NBUF)],
        [pltpu.SemaphoreType.DMA for _ in range(NBUF)],
        pltpu.SemaphoreType.DMA,
    ],
)
def _scatter_stage(data_hbm, ids_hbm, zrows_hbm, ones_hbm, zcnt_hbm,
                   psums_hbm, pcnts_hbm,
                   ids_v, rows_v, ones_v, ssum, scnt, sems, isems, zsem):
    cid = lax.axis_index("c")
    sid = lax.axis_index("s")
    wid = sid * NC + cid
    row0 = wid * ROWS_PER_TILE
    id0 = wid * NCHUNK
    seg0 = sid * SEG_PER_TILE

    def fire(j, b):
        pltpu.async_copy(data_hbm.at[pl.ds(row0 + j * CHUNK, CHUNK)],
                         rows_v[b], sems[b])
        pltpu.async_copy(ids_hbm.at[pl.ds(id0 + j, 1)], ids_v[b], isems[b])

    for j in range(NBUF):
        fire(j, j)
    pltpu.sync_copy(ones_hbm, ones_v)

    zcopies = []
    for off, sz in EXCHUNKS:
        zcopies.append(pltpu.async_copy(
            zrows_hbm.at[pl.ds(0, sz)], ssum.at[pl.ds(seg0 + off, sz)],
            zsem))
        zcopies.append(pltpu.async_copy(
            zcnt_hbm.at[pl.ds(0, sz)], scnt.at[pl.ds(seg0 + off, sz)],
            zsem))
    for c in zcopies:
        c.wait()
    plsc.subcore_barrier()

    def consume(j, b):
        pltpu.make_async_copy(data_hbm.at[pl.ds(row0, CHUNK)],
                              rows_v[b], sems[b]).wait()
        pltpu.make_async_copy(ids_hbm.at[pl.ds(id0, 1)],
                              ids_v[b], isems[b]).wait()
        pltpu.sync_copy(rows_v[b], ssum.at[ids_v[b].at[0]], add=True)
        pltpu.sync_copy(ones_v, scnt.at[ids_v[b].at[0]], add=True)

    consume(0, 0)
    fire(NBUF, 0)

    def step(i, carry):
        for b in range(NBUF):
            j = NBUF * i + 1 + b
            bb = (1 + b) % NBUF
            consume(j, bb)

            @pl.when(j + NBUF < NCHUNK)
            def _():
                fire_j = j + NBUF
                pltpu.async_copy(
                    data_hbm.at[pl.ds(row0 + fire_j * CHUNK, CHUNK)],
                    rows_v[bb], sems[bb])
                pltpu.async_copy(ids_hbm.at[pl.ds(id0 + fire_j, 1)],
                                 ids_v[bb], isems[bb])
        return carry

    lax.fori_loop(0, (NCHUNK - 1) // NBUF, step, 0)
    plsc.subcore_barrier()

    out0 = cid * NSEG + seg0
    xcopies = []
    for off, sz in EXCHUNKS:
        xcopies.append(pltpu.async_copy(
            ssum.at[pl.ds(seg0 + off, sz)],
            psums_hbm.at[pl.ds(out0 + off, sz)], zsem))
        xcopies.append(pltpu.async_copy(
            scnt.at[pl.ds(seg0 + off, sz)],
            pcnts_hbm.at[pl.ds(out0 + off, sz)], zsem))
    for c in xcopies:
        c.wait()


_FR = 1000


def _fin_body(s_ref, c_ref, o_ref):
    s = s_ref[0] + s_ref[1]
    c = c_ref[0] + c_ref[1]
    cnt = jnp.sum(c, axis=1, keepdims=True)
    o_ref[...] = s / jnp.maximum(cnt, 1.0)


_finalize = pl.pallas_call(
    _fin_body,
    grid=(NSEG // _FR,),
    in_specs=[
        pl.BlockSpec((NC, _FR, D), lambda g: (0, g, 0)),
        pl.BlockSpec((NC, _FR, CW), lambda g: (0, g, 0)),
    ],
    out_specs=pl.BlockSpec((_FR, D), lambda g: (g, 0)),
    out_shape=jax.ShapeDtypeStruct((NSEG, D), jnp.float32),
)


@jax.jit
def kernel(data, segment_ids):
    ids = segment_ids.astype(jnp.int32).reshape(NW * NCHUNK, CHUNK)
    zrows = jnp.zeros((CHUNK, D), jnp.float32)
    ones = jnp.full((CHUNK, CW), 1.0 / CW, jnp.float32)
    zcnt = jnp.zeros((CHUNK, CW), jnp.float32)
    psums, pcnts = _scatter_stage(data, ids, zrows, ones, zcnt)
    return _finalize(psums.reshape(NC, NSEG, D), pcnts.reshape(NC, NSEG, CW))


# speedup vs baseline: 1.0468x; 1.0468x over previous
import functools

import jax
import jax.numpy as jnp
from jax import lax
from jax.experimental import pallas as pl
from jax.experimental.pallas import tpu as pltpu
from jax.experimental.pallas import tpu_sc as plsc

NSEG = 10000
D = 128
N = 320000
NC = 2
NS = 16
NW = NC * NS
ROWS_PER_TILE = N // NW
CHUNK = 100
NCHUNK = ROWS_PER_TILE // CHUNK
NBUF = 3
SEG_PER_TILE = NSEG // NS
CW = 16
EXCHUNKS = [(0, 100), (100, 100), (200, 100), (300, 100),
            (400, 100), (500, 100), (600, 25)]

_mesh = plsc.VectorSubcoreMesh(core_axis_name="c", subcore_axis_name="s")


@functools.partial(
    pl.kernel,
    mesh=_mesh,
    compiler_params=pltpu.CompilerParams(use_tc_tiling_on_sc=False),
    out_type=[
        jax.ShapeDtypeStruct((NC * NSEG, D), jnp.float32),
        jax.ShapeDtypeStruct((NC * NSEG, CW), jnp.float32),
    ],
    scratch_types=[
        [pltpu.VMEM((1, CHUNK), jnp.int32) for _ in range(NBUF)],
        [pltpu.VMEM((CHUNK, D), jnp.float32) for _ in range(NBUF)],
        pltpu.VMEM((CHUNK, CW), jnp.float32),
        pltpu.VMEM_SHARED((NSEG, D), jnp.float32),
        pltpu.VMEM_SHARED((NSEG, CW), jnp.float32),
        [pltpu.SemaphoreType.DMA for _ in range(NBUF)],
        [pltpu.SemaphoreType.DMA for _ in range(NBUF)],
        pltpu.SemaphoreType.DMA,
    ],
)
def _scatter_stage(data_hbm, ids_hbm, zrows_hbm, ones_hbm, zcnt_hbm,
                   psums_hbm, pcnts_hbm,
                   ids_v, rows_v, ones_v, ssum, scnt, sems, isems, zsem):
    cid = lax.axis_index("c")
    sid = lax.axis_index("s")
    wid = sid * NC + cid
    row0 = wid * ROWS_PER_TILE
    id0 = wid * NCHUNK
    seg0 = sid * SEG_PER_TILE

    def fire(j, b):
        pltpu.async_copy(data_hbm.at[pl.ds(row0 + j * CHUNK, CHUNK)],
                         rows_v[b], sems[b])
        pltpu.async_copy(ids_hbm.at[pl.ds(id0 + j, 1)], ids_v[b], isems[b])

    for j in range(1, NBUF):
        fire(j, j)

    pltpu.sync_copy(zrows_hbm, rows_v[0])
    for off, sz in EXCHUNKS:
        pltpu.sync_copy(rows_v[0].at[pl.ds(0, sz)],
                        ssum.at[pl.ds(seg0 + off, sz)])
    fire(0, 0)
    pltpu.sync_copy(zcnt_hbm, ones_v)
    for off, sz in EXCHUNKS:
        pltpu.sync_copy(ones_v.at[pl.ds(0, sz)],
                        scnt.at[pl.ds(seg0 + off, sz)])
    pltpu.sync_copy(ones_hbm, ones_v)
    plsc.subcore_barrier()

    def consume(j, b):
        pltpu.make_async_copy(data_hbm.at[pl.ds(row0, CHUNK)],
                              rows_v[b], sems[b]).wait()
        pltpu.make_async_copy(ids_hbm.at[pl.ds(id0, 1)],
                              ids_v[b], isems[b]).wait()
        pltpu.sync_copy(rows_v[b], ssum.at[ids_v[b].at[0]], add=True)
        pltpu.sync_copy(ones_v, scnt.at[ids_v[b].at[0]], add=True)

    consume(0, 0)
    fire(NBUF, 0)

    def step(i, carry):
        for b in range(NBUF):
            j = NBUF * i + 1 + b
            bb = (1 + b) % NBUF
            consume(j, bb)

            @pl.when(j + NBUF < NCHUNK)
            def _():
                fire_j = j + NBUF
                pltpu.async_copy(
                    data_hbm.at[pl.ds(row0 + fire_j * CHUNK, CHUNK)],
                    rows_v[bb], sems[bb])
                pltpu.async_copy(ids_hbm.at[pl.ds(id0 + fire_j, 1)],
                                 ids_v[bb], isems[bb])
        return carry

    lax.fori_loop(0, (NCHUNK - 1) // NBUF, step, 0)
    plsc.subcore_barrier()

    out0 = cid * NSEG + seg0
    for off, sz in EXCHUNKS:
        pltpu.sync_copy(ssum.at[pl.ds(seg0 + off, sz)],
                        rows_v[0].at[pl.ds(0, sz)])
        pltpu.sync_copy(rows_v[0].at[pl.ds(0, sz)],
                        psums_hbm.at[pl.ds(out0 + off, sz)])
    for off, sz in EXCHUNKS:
        pltpu.sync_copy(scnt.at[pl.ds(seg0 + off, sz)],
                        ones_v.at[pl.ds(0, sz)])
        pltpu.sync_copy(ones_v.at[pl.ds(0, sz)],
                        pcnts_hbm.at[pl.ds(out0 + off, sz)])


_FR = 1000


def _fin_body(s_ref, c_ref, o_ref):
    s = s_ref[0] + s_ref[1]
    c = c_ref[0] + c_ref[1]
    cnt = jnp.sum(c, axis=1, keepdims=True)
    o_ref[...] = s / jnp.maximum(cnt, 1.0)


_finalize = pl.pallas_call(
    _fin_body,
    grid=(NSEG // _FR,),
    in_specs=[
        pl.BlockSpec((NC, _FR, D), lambda g: (0, g, 0)),
        pl.BlockSpec((NC, _FR, CW), lambda g: (0, g, 0)),
    ],
    out_specs=pl.BlockSpec((_FR, D), lambda g: (g, 0)),
    out_shape=jax.ShapeDtypeStruct((NSEG, D), jnp.float32),
)


@jax.jit
def kernel(data, segment_ids):
    ids = segment_ids.astype(jnp.int32).reshape(NW * NCHUNK, CHUNK)
    zrows = jnp.zeros((CHUNK, D), jnp.float32)
    ones = jnp.full((CHUNK, CW), 1.0 / CW, jnp.float32)
    zcnt = jnp.zeros((CHUNK, CW), jnp.float32)
    psums, pcnts = _scatter_stage(data, ids, zrows, ones, zcnt)
    return _finalize(psums.reshape(NC, NSEG, D), pcnts.reshape(NC, NSEG, CW))


# speedup vs baseline: 1.1461x; 1.0949x over previous
import functools

import jax
import jax.numpy as jnp
from jax import lax
from jax.experimental import pallas as pl
from jax.experimental.pallas import tpu as pltpu
from jax.experimental.pallas import tpu_sc as plsc

NSEG = 10000
D = 128
N = 320000
NC = 2
NS = 16
NW = NC * NS
ROWS_PER_TILE = N // NW
CHUNK = 80
NCHUNK = ROWS_PER_TILE // CHUNK
NBUF = 3
PRO = 2
SEG_PER_TILE = NSEG // NS
CW = 16
NHROW = NSEG // CW
EXCHUNKS = [(k * 80, 80) for k in range(7)] + [(560, 65)]

_mesh = plsc.VectorSubcoreMesh(core_axis_name="c", subcore_axis_name="s")


@functools.partial(
    pl.kernel,
    mesh=_mesh,
    compiler_params=pltpu.CompilerParams(use_tc_tiling_on_sc=False,
                                         needs_layout_passes=False),
    out_type=[
        jax.ShapeDtypeStruct((NC * NSEG, D), jnp.float32),
        jax.ShapeDtypeStruct((NC * NHROW, CW), jnp.float32),
    ],
    scratch_types=[
        [pltpu.VMEM((CHUNK,), jnp.int32) for _ in range(NBUF)],
        [pltpu.VMEM((CHUNK, D), jnp.float32) for _ in range(NBUF)],
        pltpu.VMEM((NHROW, CW), jnp.float32),
        pltpu.VMEM((5, 125), jnp.int32),
        pltpu.VMEM_SHARED((NSEG, D), jnp.float32),
        pltpu.VMEM_SHARED((NHROW, CW), jnp.float32),
        [pltpu.SemaphoreType.DMA for _ in range(NBUF)],
        [pltpu.SemaphoreType.DMA for _ in range(NBUF)],
    ],
)
def _scatter_stage(data_hbm, ids_hbm, zrows_hbm, zcnt_hbm, iidx_hbm,
                   psums_hbm, pcnts_hbm,
                   ids_v, rows_v, hist_v, iidx_v, ssum, scnt, sems, isems):
    cid = lax.axis_index("c")
    sid = lax.axis_index("s")
    wid = sid * NC + cid
    row0 = wid * ROWS_PER_TILE
    id0 = wid * ROWS_PER_TILE
    seg0 = sid * SEG_PER_TILE

    def fire(j, b):
        pltpu.async_copy(data_hbm.at[pl.ds(row0 + j * CHUNK, CHUNK)],
                         rows_v[b], sems[b])
        pltpu.async_copy(ids_hbm.at[pl.ds(id0 + j * CHUNK, CHUNK)],
                         ids_v[b], isems[b])

    for j in range(1, NBUF):
        fire(j, j)

    pltpu.sync_copy(zrows_hbm, rows_v[0])
    for off, sz in EXCHUNKS:
        pltpu.sync_copy(rows_v[0].at[pl.ds(0, sz)],
                        ssum.at[pl.ds(seg0 + off, sz)])
    fire(0, 0)
    pltpu.sync_copy(iidx_hbm, iidx_v)
    for k in range(5):
        pltpu.sync_copy(zcnt_hbm, hist_v.at[pl.ds(k * 125, 125)])

    @pl.when(sid == 0)
    def _():
        for k in range(5):
            pltpu.sync_copy(zcnt_hbm, scnt.at[pl.ds(k * 125, 125)])

    plsc.subcore_barrier()

    ones16 = jnp.full((16,), 1.0, jnp.float32)

    def consume(j, b):
        pltpu.make_async_copy(data_hbm.at[pl.ds(row0, CHUNK)],
                              rows_v[b], sems[b]).wait()
        pltpu.make_async_copy(ids_hbm.at[pl.ds(id0, CHUNK)],
                              ids_v[b], isems[b]).wait()
        pltpu.sync_copy(rows_v[b], ssum.at[ids_v[b]], add=True)
        for k in range(CHUNK // 16):
            idx = ids_v[b][pl.ds(k * 16, 16)]
            plsc.addupdate_scatter(hist_v, [idx >> 4, idx & 15], ones16)

    for j in range(PRO):
        consume(j, j % NBUF)
        fire(j + NBUF, j % NBUF)

    def step(i, carry):
        for b in range(NBUF):
            j = NBUF * i + PRO + b
            bb = (PRO + b) % NBUF
            consume(j, bb)

            @pl.when(j + NBUF < NCHUNK)
            def _():
                fire_j = j + NBUF
                pltpu.async_copy(
                    data_hbm.at[pl.ds(row0 + fire_j * CHUNK, CHUNK)],
                    rows_v[bb], sems[bb])
                pltpu.async_copy(
                    ids_hbm.at[pl.ds(id0 + fire_j * CHUNK, CHUNK)],
                    ids_v[bb], isems[bb])
        return carry

    lax.fori_loop(0, (NCHUNK - PRO) // NBUF, step, 0)

    for k in range(5):
        pltpu.sync_copy(hist_v.at[pl.ds(k * 125, 125)],
                        scnt.at[iidx_v.at[k]], add=True)
    plsc.subcore_barrier()

    out0 = cid * NSEG + seg0
    for off, sz in EXCHUNKS:
        pltpu.sync_copy(ssum.at[pl.ds(seg0 + off, sz)],
                        rows_v[0].at[pl.ds(0, sz)])
        pltpu.sync_copy(rows_v[0].at[pl.ds(0, sz)],
                        psums_hbm.at[pl.ds(out0 + off, sz)])

    @pl.when(sid == 0)
    def _():
        pltpu.sync_copy(scnt, hist_v)
        pltpu.sync_copy(hist_v, pcnts_hbm.at[pl.ds(cid * NHROW, NHROW)])


_FR = 2000
_FH = _FR // CW


def _fin_body(s_ref, c_ref, o_ref):
    s = s_ref[0] + s_ref[1]
    c = c_ref[0, 0] + c_ref[1, 0]
    cnt = jnp.broadcast_to(c[:, :, None], (_FH, CW, D)).reshape(_FR, D)
    o_ref[...] = s / jnp.maximum(cnt, 1.0)


_finalize = pl.pallas_call(
    _fin_body,
    grid=(NSEG // _FR,),
    in_specs=[
        pl.BlockSpec((NC, _FR, D), lambda g: (0, g, 0)),
        pl.BlockSpec((NC, 1, _FH, CW), lambda g: (0, g, 0, 0)),
    ],
    out_specs=pl.BlockSpec((_FR, D), lambda g: (g, 0)),
    out_shape=jax.ShapeDtypeStruct((NSEG, D), jnp.float32),
)


@jax.jit
def kernel(data, segment_ids):
    ids = segment_ids.astype(jnp.int32)
    zrows = jnp.zeros((CHUNK, D), jnp.float32)
    zcnt = jnp.zeros((125, CW), jnp.float32)
    iidx = jnp.arange(NHROW, dtype=jnp.int32).reshape(5, 125)
    psums, pcnts = _scatter_stage(data, ids, zrows, zcnt, iidx)
    return _finalize(psums.reshape(NC, NSEG, D),
                     pcnts.reshape(NC, NHROW // _FH, _FH, CW))
